# baseline (speedup 1.0000x reference)
import functools

import jax
import jax.numpy as jnp
from jax.experimental import pallas as pl
from jax.experimental.pallas import tpu as pltpu

_NB = 4
_BM = 1024
_BN = 1024
_SUB = 256
_SLAB = 8
_NEG = -3.0e38


def _fold(sim, t1, t2, t3):
    for c in range(sim.shape[0] // _SLAB):
        v = sim[c * _SLAB:(c + 1) * _SLAB, :]
        a = jnp.maximum(t1, v)
        v = jnp.minimum(t1, v)
        t1 = a
        a = jnp.maximum(t2, v)
        v = jnp.minimum(t2, v)
        t2 = a
        t3 = jnp.maximum(t3, v)
    return t1, t2, t3


def _mb_kernel(q_ref, b_ref, out_ref, qbf_ref, rn_ref, u1_ref, u2_ref, u3_ref):
    j = pl.program_id(1)
    nj = pl.num_programs(1)

    @pl.when(j == 0)
    def _init():
        for p in range(_NB):
            qf = q_ref[p]
            norm = jnp.sqrt(jnp.sum(qf * qf, axis=0, keepdims=True))
            rn_ref[:, p * _BM:(p + 1) * _BM] = 1.0 / jnp.maximum(norm, 1e-12)
            qbf_ref[:, p * _BM:(p + 1) * _BM] = qf.astype(jnp.bfloat16)
        neg = jnp.full(u1_ref.shape, _NEG, jnp.float32).astype(jnp.bfloat16)
        u1_ref[...] = neg
        u2_ref[...] = neg
        u3_ref[...] = neg

    qbf = qbf_ref[...]
    t1, t2, t3 = u1_ref[...], u2_ref[...], u3_ref[...]
    for s in range(_BN // _SUB):
        sub = jax.lax.dot_general(
            b_ref[s * _SUB:(s + 1) * _SUB, :].astype(jnp.bfloat16), qbf,
            dimension_numbers=(((1,), (0,)), ((), ())),
            preferred_element_type=jnp.float32,
        )
        t1, t2, t3 = _fold(sub.astype(jnp.bfloat16), t1, t2, t3)
    u1_ref[...] = t1
    u2_ref[...] = t2
    u3_ref[...] = t3

    @pl.when(j == nj - 1)
    def _finish():
        x = jnp.concatenate([t1, t2, t3], axis=0).astype(jnp.float32)
        ids = jax.lax.broadcasted_iota(jnp.int32, x.shape, 0)
        m1 = jnp.max(x, axis=0, keepdims=True)
        i1 = jnp.min(jnp.where(x == m1, ids, x.shape[0]), axis=0, keepdims=True)
        x = jnp.where(ids == i1, _NEG, x)
        m2 = jnp.max(x, axis=0, keepdims=True)
        i2 = jnp.min(jnp.where(x == m2, ids, x.shape[0]), axis=0, keepdims=True)
        x = jnp.where(ids == i2, _NEG, x)
        m3 = jnp.max(x, axis=0, keepdims=True)
        s = (3.0 - (m1 + m2 + m3) * rn_ref[...]) * 0.5
        out_ref[...] = s.reshape(_NB, 1, _BM)


@functools.partial(jax.jit, static_argnames=())
def _mb_call(qr, bank):
    nb, c, m = qr.shape
    n = bank.shape[0]
    grid = (nb // _NB, n // _BN)
    bm = _NB * _BM
    return pl.pallas_call(
        _mb_kernel,
        grid=grid,
        in_specs=[
            pl.BlockSpec((_NB, c, _BM), lambda i, j: (i, 0, 0)),
            pl.BlockSpec((_BN, c), lambda i, j: (j, 0)),
        ],
        out_specs=pl.BlockSpec((_NB, 1, _BM), lambda i, j: (i, 0, 0)),
        out_shape=jax.ShapeDtypeStruct((nb, 1, _BM), jnp.float32),
        scratch_shapes=[
            pltpu.VMEM((c, bm), jnp.bfloat16),
            pltpu.VMEM((1, bm), jnp.float32),
            pltpu.VMEM((_SLAB, bm), jnp.bfloat16),
            pltpu.VMEM((_SLAB, bm), jnp.bfloat16),
            pltpu.VMEM((_SLAB, bm), jnp.bfloat16),
        ],
        compiler_params=pltpu.CompilerParams(
            dimension_semantics=("parallel", "arbitrary"),
        ),
    )(qr, bank)


def kernel(query_features, bank_features, k):
    b, c, h, w = query_features.shape
    qr = query_features.reshape(b, c, h * w)
    dist_sum = _mb_call(qr, bank_features)
    scores = jnp.clip(dist_sum / k, 0.0, 1.0)
    return scores.reshape(b, 1, h, w)


# speedup vs baseline: 1.0072x; 1.0072x over previous
import functools

import jax
import jax.numpy as jnp
from jax.experimental import pallas as pl
from jax.experimental.pallas import tpu as pltpu

_NB = 4
_BM = 1024
_BN = 1024
_SUB = 1024
_SLAB = 8
_NEG = -3.0e38


def _fold(sim, t1, t2, t3):
    for c in range(sim.shape[0] // _SLAB):
        v = sim[c * _SLAB:(c + 1) * _SLAB, :]
        a = jnp.maximum(t1, v)
        v = jnp.minimum(t1, v)
        t1 = a
        a = jnp.maximum(t2, v)
        v = jnp.minimum(t2, v)
        t2 = a
        t3 = jnp.maximum(t3, v)
    return t1, t2, t3


def _mb_kernel(q_ref, b_ref, out_ref, qbf_ref, rn_ref, u1_ref, u2_ref, u3_ref):
    j = pl.program_id(1)
    nj = pl.num_programs(1)

    @pl.when(j == 0)
    def _init():
        for p in range(_NB):
            qf = q_ref[p]
            norm = jnp.sqrt(jnp.sum(qf * qf, axis=0, keepdims=True))
            rn_ref[:, p * _BM:(p + 1) * _BM] = 1.0 / jnp.maximum(norm, 1e-12)
            qbf_ref[:, p * _BM:(p + 1) * _BM] = qf.astype(jnp.bfloat16)
        neg = jnp.full(u1_ref.shape, _NEG, jnp.float32).astype(jnp.bfloat16)
        u1_ref[...] = neg
        u2_ref[...] = neg
        u3_ref[...] = neg

    qbf = qbf_ref[...]
    t1, t2, t3 = u1_ref[...], u2_ref[...], u3_ref[...]
    for s in range(_BN // _SUB):
        sub = jax.lax.dot_general(
            b_ref[s * _SUB:(s + 1) * _SUB, :].astype(jnp.bfloat16), qbf,
            dimension_numbers=(((1,), (0,)), ((), ())),
            preferred_element_type=jnp.float32,
        )
        t1, t2, t3 = _fold(sub.astype(jnp.bfloat16), t1, t2, t3)
    u1_ref[...] = t1
    u2_ref[...] = t2
    u3_ref[...] = t3

    @pl.when(j == nj - 1)
    def _finish():
        x = jnp.concatenate([t1, t2, t3], axis=0).astype(jnp.float32)
        ids = jax.lax.broadcasted_iota(jnp.int32, x.shape, 0)
        m1 = jnp.max(x, axis=0, keepdims=True)
        i1 = jnp.min(jnp.where(x == m1, ids, x.shape[0]), axis=0, keepdims=True)
        x = jnp.where(ids == i1, _NEG, x)
        m2 = jnp.max(x, axis=0, keepdims=True)
        i2 = jnp.min(jnp.where(x == m2, ids, x.shape[0]), axis=0, keepdims=True)
        x = jnp.where(ids == i2, _NEG, x)
        m3 = jnp.max(x, axis=0, keepdims=True)
        s = (3.0 - (m1 + m2 + m3) * rn_ref[...]) * 0.5
        out_ref[...] = s.reshape(_NB, 1, _BM)


@functools.partial(jax.jit, static_argnames=())
def _mb_call(qr, bank):
    nb, c, m = qr.shape
    n = bank.shape[0]
    grid = (nb // _NB, n // _BN)
    bm = _NB * _BM
    return pl.pallas_call(
        _mb_kernel,
        grid=grid,
        in_specs=[
            pl.BlockSpec((_NB, c, _BM), lambda i, j: (i, 0, 0)),
            pl.BlockSpec((_BN, c), lambda i, j: (j, 0)),
        ],
        out_specs=pl.BlockSpec((_NB, 1, _BM), lambda i, j: (i, 0, 0)),
        out_shape=jax.ShapeDtypeStruct((nb, 1, _BM), jnp.float32),
        scratch_shapes=[
            pltpu.VMEM((c, bm), jnp.bfloat16),
            pltpu.VMEM((1, bm), jnp.float32),
            pltpu.VMEM((_SLAB, bm), jnp.bfloat16),
            pltpu.VMEM((_SLAB, bm), jnp.bfloat16),
            pltpu.VMEM((_SLAB, bm), jnp.bfloat16),
        ],
        compiler_params=pltpu.CompilerParams(
            dimension_semantics=("parallel", "arbitrary"),
        ),
    )(qr, bank)


def kernel(query_features, bank_features, k):
    b, c, h, w = query_features.shape
    qr = query_features.reshape(b, c, h * w)
    dist_sum = _mb_call(qr, bank_features)
    scores = jnp.clip(dist_sum / k, 0.0, 1.0)
    return scores.reshape(b, 1, h, w)
